# baseline (speedup 1.0000x reference)
import jax
import jax.numpy as jnp
from jax.experimental import pallas as pl
from jax.experimental.pallas import tpu as pltpu
from functools import partial

W_CLASS = 2.0
W_L1 = 5.0
W_GIOU = 2.0
OTA_K = 5
ALPHA = 0.25
GAMMA = 2.0
CENTER_RADIUS = 2.5
IMG = 1024.0

_INF = float("inf")
_IBIG = 1 << 30


def _extract5(vals, idxs):
    outv, outi = [], []
    W, Wi = vals, idxs
    for _ in range(OTA_K):
        m = jnp.min(W, axis=0, keepdims=True)
        sel = W == m
        mi = jnp.min(jnp.where(sel, Wi, _IBIG), axis=0, keepdims=True)
        outv.append(m)
        outi.append(mi)
        W = jnp.where(sel & (Wi == mi), _INF, W)
    outv.append(jnp.full((3, 128), _INF, jnp.float32))
    outi.append(jnp.full((3, 128), _IBIG, jnp.int32))
    return jnp.concatenate(outv, axis=0), jnp.concatenate(outi, axis=0)


def _tile_kernel(x1_ref, y1_ref, x2_ref, y2_ref, cls_ref, gt_ref,
                 cost_ref, iou5_ref, lc5v_ref, lc5i_ref,
                 niov_acc, cv_acc, ci_acc, *, tn, g):
    t = pl.program_id(1)

    @pl.when(t == 0)
    def _init():
        niov_acc[...] = jnp.full((8, 128), _INF, jnp.float32)
        cv_acc[...] = jnp.full((8, 128), _INF, jnp.float32)
        ci_acc[...] = jnp.full((8, 128), _IBIG, jnp.int32)

    gt = gt_ref[0]
    gx1 = gt[0:1, :]
    gy1 = gt[1:2, :]
    gx2 = gt[2:3, :]
    gy2 = gt[3:4, :]
    cxl = gt[4:5, :]
    cxh = gt[5:6, :]
    cyl = gt[6:7, :]
    cyh = gt[7:8, :]
    gn1 = gt[8:9, :]
    gn2 = gt[9:10, :]
    gn3 = gt[10:11, :]
    gn4 = gt[11:12, :]
    area_b = gt[12:13, :]

    lane_pad = jax.lax.broadcasted_iota(jnp.int32, (1, 128), 1) >= g
    iota8 = jax.lax.broadcasted_iota(jnp.int32, (8, 128), 0)
    inv = 1.0 / IMG

    rv = [jnp.full((8, 128), _INF, jnp.float32) for _ in range(OTA_K)]
    ri = [jnp.full((8, 128), _IBIG, jnp.int32) for _ in range(OTA_K)]
    uv = [jnp.full((8, 128), _INF, jnp.float32) for _ in range(OTA_K)]

    for s in range(tn // 8):
        sl = slice(8 * s, 8 * s + 8)
        x1 = x1_ref[0, sl, :]
        y1 = y1_ref[0, sl, :]
        x2 = x2_ref[0, sl, :]
        y2 = y2_ref[0, sl, :]
        cls_c = cls_ref[0, sl, :]

        px = (x1 + x2) * 0.5
        py = (y1 + y2) * 0.5
        strict = (px > gx1) & (px < gx2) & (py > gy1) & (py < gy2)
        circ = (px > cxl) & (px < cxh) & (py > cyl) & (py < cyh)
        loose = jnp.any(circ, axis=1, keepdims=True)

        d0 = jnp.abs(x1 * inv - gn1)
        d1 = jnp.abs(y1 * inv - gn2)
        d2 = jnp.abs(x2 * inv - gn3)
        d3 = jnp.abs(y2 * inv - gn4)
        cost_bbox = ((d0 + d1) + d2) + d3

        area_a = (x2 - x1) * (y2 - y1)
        iw = jnp.clip(jnp.minimum(x2, gx2) - jnp.maximum(x1, gx1), 0.0)
        ih = jnp.clip(jnp.minimum(y2, gy2) - jnp.maximum(y1, gy1), 0.0)
        inter = iw * ih
        union = area_a + area_b - inter
        iou = inter / (union + 1e-8)
        ew = jnp.clip(jnp.maximum(x2, gx2) - jnp.minimum(x1, gx1), 0.0)
        eh = jnp.clip(jnp.maximum(y2, gy2) - jnp.minimum(y1, gy1), 0.0)
        earea = ew * eh
        giou = iou - (earea - union) / (earea + 1e-8)

        cost = (W_L1 * cost_bbox + W_CLASS * cls_c + W_GIOU * (-giou)
                + jnp.where(strict, 0.0, 100.0))
        cost = cost + jnp.where(loose, 0.0, 10000.0)
        cost = jnp.where(lane_pad, _INF, cost)
        cost_ref[0, sl, :] = cost

        X = cost
        Xi = iota8 + (t * tn + 8 * s)
        for l in range(OTA_K):
            c = X < rv[l]
            nv_ = jnp.where(c, X, rv[l])
            ni_ = jnp.where(c, Xi, ri[l])
            X = jnp.where(c, rv[l], X)
            Xi = jnp.where(c, ri[l], Xi)
            rv[l], ri[l] = nv_, ni_

        Y = -iou
        for l in range(OTA_K):
            nu_ = jnp.minimum(Y, uv[l])
            Y = jnp.maximum(Y, uv[l])
            uv[l] = nu_

    wv = jnp.concatenate(rv + [cv_acc[...]], axis=0)
    wi = jnp.concatenate(ri + [ci_acc[...]], axis=0)
    nv, ni = _extract5(wv, wi)
    cv_acc[...] = nv
    ci_acc[...] = ni

    wv = jnp.concatenate(uv + [niov_acc[...]], axis=0)
    wi = jax.lax.broadcasted_iota(jnp.int32, (OTA_K * 8 + 8, 128), 0)
    nv, ni = _extract5(wv, wi)
    niov_acc[...] = nv

    iou5_ref[0] = -niov_acc[...]
    lc5v_ref[0] = cv_acc[...]
    lc5i_ref[0] = ci_acc[...]


def _two_sum(a, b):
    s = a + b
    bb = s - a
    err = (a - (s - bb)) + (b - bb)
    return s, err


def _dynamic_ks(iou5):
    v = [iou5[:, i, :] for i in range(OTA_K)]
    for _ in range(OTA_K):
        for i in range(1, OTA_K):
            s, e = _two_sum(v[i - 1], v[i])
            v[i - 1], v[i] = e, s
    hi = v[-1]
    r = v[0]
    for i in range(1, OTA_K - 1):
        r = r + v[i]
    m = jnp.floor(hi)
    frac = hi - m
    k = (m + jnp.where((frac - 1.0) + r >= 0.0, 1.0, 0.0)
         - jnp.where((frac == 0.0) & (r < 0.0), 1.0, 0.0))
    return jnp.maximum(k.astype(jnp.int32), 1)


def _matching(cost, iou5, lv, li, B, N, G):
    L = 128
    E = OTA_K * L
    ks = _dynamic_ks(iou5)
    lane = jnp.arange(L, dtype=jnp.int32)
    krow = jnp.arange(OTA_K, dtype=jnp.int32)
    valid = ((krow[None, :, None] < ks[:, None, :])
             & (lane[None, None, :] < G)).reshape(B, E)
    q = li.reshape(B, E)
    c = lv.reshape(B, E)
    gl = jnp.tile(lane, OTA_K)

    better = (valid[:, None, :]
              & (q[:, :, None] == q[:, None, :])
              & ((c[:, None, :] < c[:, :, None])
                 | ((c[:, None, :] == c[:, :, None])
                    & (gl[None, None, :] < gl[None, :, None]))))
    winner = valid & ~jnp.any(better, axis=2)
    counts = winner.reshape(B, OTA_K, L).sum(axis=1).astype(jnp.int32)
    counts = jnp.where(lane[None, :] < G, counts, 1)

    brow = jnp.arange(B, dtype=jnp.int32)[:, None]
    a = jnp.full((B, N), -1, jnp.int32).at[brow, q].max(
        jnp.where(winner, gl[None, :], -1), mode="drop")

    def cond_fn(state):
        _, counts = state
        return jnp.any(counts == 0)

    def body_fn(state):
        a, counts = state
        matched = a >= 0
        costm = jnp.where(matched[:, :, None], _INF, cost)
        pos = jnp.argmin(costm, axis=1)
        posc = jnp.take_along_axis(costm, pos[:, None, :], axis=1)[:, 0, :]
        zero = counts == 0
        better2 = (zero[:, None, :]
                   & (pos[:, :, None] == pos[:, None, :])
                   & ((posc[:, None, :] < posc[:, :, None])
                      | ((posc[:, None, :] == posc[:, :, None])
                         & (lane[None, None, :] < lane[None, :, None]))))
        win = zero & ~jnp.any(better2, axis=2)
        a = a.at[brow, pos].max(jnp.where(win, lane[None, :], -1),
                                mode="drop")
        counts = counts + win.astype(jnp.int32)
        return a, counts

    a, counts = jax.lax.while_loop(cond_fn, body_fn, (a, counts))
    return a >= 0, a


def kernel(pred_logits, pred_xyxy, gt_cxcywh):
    B, N = pred_logits.shape[0], pred_logits.shape[1]
    G = gt_cxcywh.shape[1]
    assert G <= 128

    tn = None
    for d in range(min(2048, N), 7, -1):
        if N % d == 0 and d % 8 == 0:
            tn = d
            break
    assert tn is not None, "N must have a divisor that is a multiple of 8"
    nt = N // tn

    p = jax.nn.sigmoid(pred_logits)
    neg_cost = (1.0 - ALPHA) * p ** GAMMA * (-jnp.log(1.0 - p + 1e-8))
    pos_cost = ALPHA * (1.0 - p) ** GAMMA * (-jnp.log(p + 1e-8))
    cls_cost = pos_cost - neg_cost

    x1 = pred_xyxy[:, :, 0:1]
    y1 = pred_xyxy[:, :, 1:2]
    x2 = pred_xyxy[:, :, 2:3]
    y2 = pred_xyxy[:, :, 3:4]

    gcx, gcy = gt_cxcywh[:, :, 0], gt_cxcywh[:, :, 1]
    gw_, gh_ = gt_cxcywh[:, :, 2], gt_cxcywh[:, :, 3]
    gx1 = gcx - 0.5 * gw_
    gy1 = gcy - 0.5 * gh_
    gx2 = gcx + 0.5 * gw_
    gy2 = gcy + 0.5 * gh_
    gw = gx2 - gx1
    gh = gy2 - gy1
    rows = [
        gx1, gy1, gx2, gy2,
        gcx - CENTER_RADIUS * gw, gcx + CENTER_RADIUS * gw,
        gcy - CENTER_RADIUS * gh, gcy + CENTER_RADIUS * gh,
        gx1 / IMG, gy1 / IMG, gx2 / IMG, gy2 / IMG,
        (gx2 - gx1) * (gy2 - gy1),
    ]
    gtrows = jnp.stack(rows, axis=1)
    pad_cols = jnp.zeros((B, 13, 128 - G), jnp.float32)
    pad_cols = pad_cols.at[:, 4, :].set(_INF)
    pad_cols = pad_cols.at[:, 6, :].set(_INF)
    gtrows = jnp.concatenate([gtrows, pad_cols], axis=2)
    gtrows = jnp.concatenate(
        [gtrows, jnp.zeros((B, 3, 128), jnp.float32)], axis=1)

    row_spec = pl.BlockSpec((1, tn, 1), lambda b, t: (b, t, 0))
    acc_spec = pl.BlockSpec((1, 8, 128), lambda b, t: (b, 0, 0))
    cost, iou5, lv, li = pl.pallas_call(
        partial(_tile_kernel, tn=tn, g=G),
        grid=(B, nt),
        in_specs=[row_spec, row_spec, row_spec, row_spec, row_spec,
                  pl.BlockSpec((1, 16, 128), lambda b, t: (b, 0, 0))],
        out_specs=[pl.BlockSpec((1, tn, 128), lambda b, t: (b, t, 0)),
                   acc_spec, acc_spec, acc_spec],
        out_shape=[jax.ShapeDtypeStruct((B, N, 128), jnp.float32),
                   jax.ShapeDtypeStruct((B, 8, 128), jnp.float32),
                   jax.ShapeDtypeStruct((B, 8, 128), jnp.float32),
                   jax.ShapeDtypeStruct((B, 8, 128), jnp.int32)],
        scratch_shapes=[pltpu.VMEM((8, 128), jnp.float32),
                        pltpu.VMEM((8, 128), jnp.float32),
                        pltpu.VMEM((8, 128), jnp.int32)],
    )(x1, y1, x2, y2, cls_cost, gtrows)

    sel, a = _matching(cost, iou5[:, :OTA_K, :], lv[:, :OTA_K, :],
                       li[:, :OTA_K, :], B, N, G)
    return sel, a


# speedup vs baseline: 1.2569x; 1.2569x over previous
import jax
import jax.numpy as jnp
from jax.experimental import pallas as pl
from jax.experimental.pallas import tpu as pltpu
from functools import partial

W_CLASS = 2.0
W_L1 = 5.0
W_GIOU = 2.0
OTA_K = 5
ALPHA = 0.25
GAMMA = 2.0
CENTER_RADIUS = 2.5
IMG = 1024.0

_INF = float("inf")
_IBIG = 1 << 30


def _extract5(vals, idxs):
    outv, outi = [], []
    W, Wi = vals, idxs
    for _ in range(OTA_K):
        m = jnp.min(W, axis=0, keepdims=True)
        sel = W == m
        mi = jnp.min(jnp.where(sel, Wi, _IBIG), axis=0, keepdims=True)
        outv.append(m)
        outi.append(mi)
        W = jnp.where(sel & (Wi == mi), _INF, W)
    outv.append(jnp.full((3, 128), _INF, jnp.float32))
    outi.append(jnp.full((3, 128), _IBIG, jnp.int32))
    return jnp.concatenate(outv, axis=0), jnp.concatenate(outi, axis=0)


def _tile_kernel(x1_ref, y1_ref, x2_ref, y2_ref, cls_ref, gt_ref,
                 cost_ref, iou5_ref, lc5v_ref, lc5i_ref,
                 niov_acc, nioi_acc, cv_acc, ci_acc, *, tn, g):
    t = pl.program_id(1)

    @pl.when(t == 0)
    def _init():
        niov_acc[...] = jnp.full((8, 128), _INF, jnp.float32)
        nioi_acc[...] = jnp.full((8, 128), _IBIG, jnp.int32)
        cv_acc[...] = jnp.full((8, 128), _INF, jnp.float32)
        ci_acc[...] = jnp.full((8, 128), _IBIG, jnp.int32)

    x1 = x1_ref[0]
    y1 = y1_ref[0]
    x2 = x2_ref[0]
    y2 = y2_ref[0]
    cls_c = cls_ref[0]
    gt = gt_ref[0]
    gx1 = gt[0:1, :]
    gy1 = gt[1:2, :]
    gx2 = gt[2:3, :]
    gy2 = gt[3:4, :]
    cxl = gt[4:5, :]
    cxh = gt[5:6, :]
    cyl = gt[6:7, :]
    cyh = gt[7:8, :]
    gn1 = gt[8:9, :]
    gn2 = gt[9:10, :]
    gn3 = gt[10:11, :]
    gn4 = gt[11:12, :]
    area_b = gt[12:13, :]

    px = (x1 + x2) * 0.5
    py = (y1 + y2) * 0.5
    strict = (px > gx1) & (px < gx2) & (py > gy1) & (py < gy2)
    circ = (px > cxl) & (px < cxh) & (py > cyl) & (py < cyh)
    loose = jnp.any(circ, axis=1, keepdims=True)

    inv = 1.0 / IMG
    d0 = jnp.abs(x1 * inv - gn1)
    d1 = jnp.abs(y1 * inv - gn2)
    d2 = jnp.abs(x2 * inv - gn3)
    d3 = jnp.abs(y2 * inv - gn4)
    cost_bbox = ((d0 + d1) + d2) + d3

    area_a = (x2 - x1) * (y2 - y1)
    iw = jnp.clip(jnp.minimum(x2, gx2) - jnp.maximum(x1, gx1), 0.0)
    ih = jnp.clip(jnp.minimum(y2, gy2) - jnp.maximum(y1, gy1), 0.0)
    inter = iw * ih
    union = area_a + area_b - inter
    iou = inter / (union + 1e-8)
    ew = jnp.clip(jnp.maximum(x2, gx2) - jnp.minimum(x1, gx1), 0.0)
    eh = jnp.clip(jnp.maximum(y2, gy2) - jnp.minimum(y1, gy1), 0.0)
    earea = ew * eh
    giou = iou - (earea - union) / (earea + 1e-8)

    cost = (W_L1 * cost_bbox + W_CLASS * cls_c + W_GIOU * (-giou)
            + jnp.where(strict, 0.0, 100.0))
    cost = cost + jnp.where(loose, 0.0, 10000.0)

    lane = jax.lax.broadcasted_iota(jnp.int32, (1, 128), 1)
    cost = jnp.where(lane < g, cost, _INF)
    cost_ref[0] = cost

    gidx = jax.lax.broadcasted_iota(jnp.int32, (tn, 128), 0) + t * tn

    wv = jnp.concatenate([cost, cv_acc[...]], axis=0)
    wi = jnp.concatenate([gidx, ci_acc[...]], axis=0)
    nv, ni = _extract5(wv, wi)
    cv_acc[...] = nv
    ci_acc[...] = ni

    wv = jnp.concatenate([-iou, niov_acc[...]], axis=0)
    wi = jnp.concatenate([gidx, nioi_acc[...]], axis=0)
    nv, ni = _extract5(wv, wi)
    niov_acc[...] = nv
    nioi_acc[...] = ni

    iou5_ref[0] = -niov_acc[...]
    lc5v_ref[0] = cv_acc[...]
    lc5i_ref[0] = ci_acc[...]


def _two_sum(a, b):
    s = a + b
    bb = s - a
    err = (a - (s - bb)) + (b - bb)
    return s, err


def _dynamic_ks(iou5):
    v = [iou5[:, i, :] for i in range(OTA_K)]
    for _ in range(OTA_K):
        for i in range(1, OTA_K):
            s, e = _two_sum(v[i - 1], v[i])
            v[i - 1], v[i] = e, s
    hi = v[-1]
    r = v[0]
    for i in range(1, OTA_K - 1):
        r = r + v[i]
    m = jnp.floor(hi)
    frac = hi - m
    k = (m + jnp.where((frac - 1.0) + r >= 0.0, 1.0, 0.0)
         - jnp.where((frac == 0.0) & (r < 0.0), 1.0, 0.0))
    return jnp.maximum(k.astype(jnp.int32), 1)


def _matching(cost, iou5, lv, li, B, N, G):
    L = 128
    E = OTA_K * L
    ks = _dynamic_ks(iou5)
    lane = jnp.arange(L, dtype=jnp.int32)
    krow = jnp.arange(OTA_K, dtype=jnp.int32)
    valid = ((krow[None, :, None] < ks[:, None, :])
             & (lane[None, None, :] < G)).reshape(B, E)
    q = li.reshape(B, E)
    c = lv.reshape(B, E)
    gl = jnp.tile(lane, OTA_K)

    better = (valid[:, None, :]
              & (q[:, :, None] == q[:, None, :])
              & ((c[:, None, :] < c[:, :, None])
                 | ((c[:, None, :] == c[:, :, None])
                    & (gl[None, None, :] < gl[None, :, None]))))
    winner = valid & ~jnp.any(better, axis=2)
    counts = winner.reshape(B, OTA_K, L).sum(axis=1).astype(jnp.int32)
    counts = jnp.where(lane[None, :] < G, counts, 1)

    brow = jnp.arange(B, dtype=jnp.int32)[:, None]
    a = jnp.full((B, N), -1, jnp.int32).at[brow, q].max(
        jnp.where(winner, gl[None, :], -1), mode="drop")

    def cond_fn(state):
        _, counts = state
        return jnp.any(counts == 0)

    def body_fn(state):
        a, counts = state
        matched = a >= 0
        costm = jnp.where(matched[:, :, None], _INF, cost)
        pos = jnp.argmin(costm, axis=1)
        posc = jnp.take_along_axis(costm, pos[:, None, :], axis=1)[:, 0, :]
        zero = counts == 0
        better2 = (zero[:, None, :]
                   & (pos[:, :, None] == pos[:, None, :])
                   & ((posc[:, None, :] < posc[:, :, None])
                      | ((posc[:, None, :] == posc[:, :, None])
                         & (lane[None, None, :] < lane[None, :, None]))))
        win = zero & ~jnp.any(better2, axis=2)
        a = a.at[brow, pos].max(jnp.where(win, lane[None, :], -1),
                                mode="drop")
        counts = counts + win.astype(jnp.int32)
        return a, counts

    a, counts = jax.lax.while_loop(cond_fn, body_fn, (a, counts))
    return a >= 0, a


def kernel(pred_logits, pred_xyxy, gt_cxcywh):
    B, N = pred_logits.shape[0], pred_logits.shape[1]
    G = gt_cxcywh.shape[1]
    assert G <= 128

    tn = None
    for d in range(min(2048, N), 7, -1):
        if N % d == 0 and d % 8 == 0:
            tn = d
            break
    assert tn is not None, "N must have a divisor that is a multiple of 8"
    nt = N // tn

    p = jax.nn.sigmoid(pred_logits)
    neg_cost = (1.0 - ALPHA) * p ** GAMMA * (-jnp.log(1.0 - p + 1e-8))
    pos_cost = ALPHA * (1.0 - p) ** GAMMA * (-jnp.log(p + 1e-8))
    cls_cost = pos_cost - neg_cost

    x1 = pred_xyxy[:, :, 0:1]
    y1 = pred_xyxy[:, :, 1:2]
    x2 = pred_xyxy[:, :, 2:3]
    y2 = pred_xyxy[:, :, 3:4]

    gcx, gcy = gt_cxcywh[:, :, 0], gt_cxcywh[:, :, 1]
    gw_, gh_ = gt_cxcywh[:, :, 2], gt_cxcywh[:, :, 3]
    gx1 = gcx - 0.5 * gw_
    gy1 = gcy - 0.5 * gh_
    gx2 = gcx + 0.5 * gw_
    gy2 = gcy + 0.5 * gh_
    gw = gx2 - gx1
    gh = gy2 - gy1
    rows = [
        gx1, gy1, gx2, gy2,
        gcx - CENTER_RADIUS * gw, gcx + CENTER_RADIUS * gw,
        gcy - CENTER_RADIUS * gh, gcy + CENTER_RADIUS * gh,
        gx1 / IMG, gy1 / IMG, gx2 / IMG, gy2 / IMG,
        (gx2 - gx1) * (gy2 - gy1),
    ]
    gtrows = jnp.stack(rows, axis=1)
    pad_cols = jnp.zeros((B, 13, 128 - G), jnp.float32)
    pad_cols = pad_cols.at[:, 4, :].set(_INF)
    pad_cols = pad_cols.at[:, 6, :].set(_INF)
    gtrows = jnp.concatenate([gtrows, pad_cols], axis=2)
    gtrows = jnp.concatenate(
        [gtrows, jnp.zeros((B, 3, 128), jnp.float32)], axis=1)

    row_spec = pl.BlockSpec((1, tn, 1), lambda b, t: (b, t, 0))
    acc_spec = pl.BlockSpec((1, 8, 128), lambda b, t: (b, 0, 0))
    cost, iou5, lv, li = pl.pallas_call(
        partial(_tile_kernel, tn=tn, g=G),
        grid=(B, nt),
        in_specs=[row_spec, row_spec, row_spec, row_spec, row_spec,
                  pl.BlockSpec((1, 16, 128), lambda b, t: (b, 0, 0))],
        out_specs=[pl.BlockSpec((1, tn, 128), lambda b, t: (b, t, 0)),
                   acc_spec, acc_spec, acc_spec],
        out_shape=[jax.ShapeDtypeStruct((B, N, 128), jnp.float32),
                   jax.ShapeDtypeStruct((B, 8, 128), jnp.float32),
                   jax.ShapeDtypeStruct((B, 8, 128), jnp.float32),
                   jax.ShapeDtypeStruct((B, 8, 128), jnp.int32)],
        scratch_shapes=[pltpu.VMEM((8, 128), jnp.float32),
                        pltpu.VMEM((8, 128), jnp.int32),
                        pltpu.VMEM((8, 128), jnp.float32),
                        pltpu.VMEM((8, 128), jnp.int32)],
    )(x1, y1, x2, y2, cls_cost, gtrows)

    sel, a = _matching(cost, iou5[:, :OTA_K, :], lv[:, :OTA_K, :],
                       li[:, :OTA_K, :], B, N, G)
    return sel, a
